# baseline (speedup 1.0000x reference)
import functools

import jax
import jax.numpy as jnp
from jax import lax
from jax.experimental import pallas as pl
from jax.experimental.pallas import tpu as pltpu
from jax.experimental.pallas import tpu_sc as plsc

BATCH = 4096
SEQ_LEN = 200
TIME_DIM = 32

NUM_CORES = 2
NUM_SUBCORES = 16
NW = NUM_CORES * NUM_SUBCORES
ROWS_PER_W = BATCH // NW
NBUF = 2


def _gather_sc(table, idx):
    mesh = plsc.VectorSubcoreMesh(core_axis_name="c", subcore_axis_name="s")

    @functools.partial(
        pl.kernel,
        mesh=mesh,
        compiler_params=pltpu.CompilerParams(use_tc_tiling_on_sc=False),
        out_type=jax.ShapeDtypeStruct((BATCH, SEQ_LEN, TIME_DIM), jnp.float32),
        scratch_types=[
            pltpu.VMEM((ROWS_PER_W, SEQ_LEN), jnp.int32),
            pltpu.VMEM((NBUF, SEQ_LEN, TIME_DIM), jnp.float32),
            pltpu.SemaphoreType.DMA((NBUF,)),
            pltpu.SemaphoreType.DMA((NBUF,)),
        ],
    )
    def k(table_hbm, idx_hbm, out_hbm, idx_all, rows, gsem, ssem):
        wid = lax.axis_index("s") * NUM_CORES + lax.axis_index("c")
        base = wid * ROWS_PER_W
        pltpu.sync_copy(idx_hbm.at[pl.ds(base, ROWS_PER_W)], idx_all)

        def g_copy(i, b):
            return pltpu.make_async_copy(
                table_hbm.at[idx_all.at[i, :]],
                rows.at[b],
                gsem.at[b],
            )

        def s_copy(i, b):
            return pltpu.make_async_copy(
                rows.at[b],
                out_hbm.at[base + i],
                ssem.at[b],
            )

        g_copy(0, 0).start()

        def body(jj, carry):
            for b in range(NBUF):
                i = jj * NBUF + b
                nb = (b + 1) % NBUF

                @pl.when(i + 1 < ROWS_PER_W)
                def _():
                    @pl.when(i >= 1)
                    def _():
                        s_copy(i - 1, nb).wait()

                    g_copy(i + 1, nb).start()

                g_copy(i, b).wait()
                s_copy(i, b).start()
            return carry

        lax.fori_loop(0, ROWS_PER_W // NBUF, body, 0)
        s_copy(ROWS_PER_W - 2, 0).wait()
        s_copy(ROWS_PER_W - 1, 1).wait()

    return k(table, idx)


def kernel(token_ids, time_embeddings):
    return _gather_sc(time_embeddings, token_ids)
